# baseline (speedup 1.0000x reference)
import functools

import jax
import jax.numpy as jnp
from jax import lax
from jax.experimental import pallas as pl
from jax.experimental.pallas import tpu as pltpu
from jax.experimental.pallas import tpu_sc as plsc

F32 = jnp.float32
I32 = jnp.int32

E0 = 160000
M0 = 320000
NA = 10000
D = 128
NB = 8
MP = 327680
EP = 163840
AP = 10240

CHK = 3200
ACC = 3328
MSL = MP // 16
NCK = 50
NCHUNK = 25

ACHK = 5120
ACA = 5376
ESL = EP // 16


def _act(x):
    return x * (1.0 / (1.0 + jnp.exp(-x)))


def _mm_nt(a, w):
    return lax.dot_general(a, w, (((1,), (1,)), ((), ())),
                           preferred_element_type=F32)


def _s1_body(mji_ref, rbf_ref, ws_ref, bs_ref, wr_ref, x_ref):
    h = _act(_mm_nt(mji_ref[...], ws_ref[...]) + bs_ref[...])
    x_ref[...] = h * _mm_nt(rbf_ref[...], wr_ref[...])


def _s3_body(xk_ref, sbf_ref, wsbf_ref, wb_ref, msg_ref):
    s = _mm_nt(sbf_ref[...], wsbf_ref[...])
    t = jnp.dot(xk_ref[...], wb_ref[...], preferred_element_type=F32)
    acc = t[:, 0:D] * s[:, 0:1]
    for l in range(1, NB):
        acc = acc + t[:, l * D:(l + 1) * D] * s[:, l:l + 1]
    msg_ref[...] = acc


def _s5_body(m0_ref, mji_ref, rbf_ref, gate_ref,
             riW1, rib1, riW2, rib2, liW, lib,
             r1W1, r1b1, r1W2, r1b2, r2W1, r2b1, r2W2, r2b2,
             wro, m_out, a_out):
    def resid(x, W1, b1, W2, b2):
        v = _act(x)
        v = _act(_mm_nt(v, W1[...]) + b1[...])
        v = _mm_nt(v, W2[...]) + b2[...]
        return x + v

    m = m0_ref[...]
    m = resid(m, riW1, rib1, riW2, rib2)
    m = _act(_mm_nt(m, liW[...]) + lib[...]) + gate_ref[...] * mji_ref[...]
    m = resid(m, r1W1, r1b1, r1W2, r1b2)
    m = resid(m, r2W1, r2b1, r2W2, r2b2)
    m_out[...] = m
    a_out[...] = m * _mm_nt(rbf_ref[...], wro[...])


def _s7_body(atom_ref, w0, b0, w1, b1, wo, bo, out_ref):
    h = _act(_mm_nt(atom_ref[...], w0[...]) + b0[...])
    h = _act(_mm_nt(h, w1[...]) + b1[...])
    out_ref[...] = _mm_nt(h, wo[...]) + bo[...]


def _s1_call(mji, rbf, Ws, bs, Wr):
    B = 2000
    n = E0 // B
    return pl.pallas_call(
        _s1_body,
        grid=(n,),
        in_specs=[pl.BlockSpec((B, D), lambda i: (i, 0)),
                  pl.BlockSpec((B, 16), lambda i: (i, 0)),
                  pl.BlockSpec((D, D), lambda i: (0, 0)),
                  pl.BlockSpec((1, D), lambda i: (0, 0)),
                  pl.BlockSpec((D, 16), lambda i: (0, 0))],
        out_specs=pl.BlockSpec((B, D), lambda i: (i, 0)),
        out_shape=jax.ShapeDtypeStruct((E0, D), F32),
    )(mji, rbf, Ws, bs, Wr)


def _s3_call(xk, sbf, Wsbf, Wb):
    B = 512
    n = MP // B
    return pl.pallas_call(
        _s3_body,
        grid=(n,),
        in_specs=[pl.BlockSpec((B, D), lambda i: (i, 0)),
                  pl.BlockSpec((B, 16), lambda i: (i, 0)),
                  pl.BlockSpec((NB, 16), lambda i: (0, 0)),
                  pl.BlockSpec((D, NB * D), lambda i: (0, 0))],
        out_specs=pl.BlockSpec((B, D), lambda i: (i, 0)),
        out_shape=jax.ShapeDtypeStruct((MP, D), F32),
    )(xk, sbf, Wsbf, Wb)


def _s5_call(m0, mji, rbf, gate, riW1, rib1, riW2, rib2, liW, lib,
             r1W1, r1b1, r1W2, r1b2, r2W1, r2b1, r2W2, r2b2, wro):
    B = 2000
    n = E0 // B
    row = lambda: pl.BlockSpec((B, D), lambda i: (i, 0))
    wmat = lambda: pl.BlockSpec((D, D), lambda i: (0, 0))
    wvec = lambda: pl.BlockSpec((1, D), lambda i: (0, 0))
    return pl.pallas_call(
        _s5_body,
        grid=(n,),
        in_specs=[row(), row(), pl.BlockSpec((B, 16), lambda i: (i, 0)), wvec(),
                  wmat(), wvec(), wmat(), wvec(), wmat(), wvec(),
                  wmat(), wvec(), wmat(), wvec(), wmat(), wvec(), wmat(), wvec(),
                  pl.BlockSpec((D, 16), lambda i: (0, 0))],
        out_specs=[row(), row()],
        out_shape=[jax.ShapeDtypeStruct((E0, D), F32),
                   jax.ShapeDtypeStruct((E0, D), F32)],
    )(m0, mji, rbf, gate, riW1, rib1, riW2, rib2, liW, lib,
      r1W1, r1b1, r1W2, r1b2, r2W1, r2b1, r2W2, r2b2, wro)


def _s7_call(atom, w0, b0, w1, b1, wo, bo):
    B = 1280
    n = AP // B
    return pl.pallas_call(
        _s7_body,
        grid=(n,),
        in_specs=[pl.BlockSpec((B, D), lambda i: (i, 0)),
                  pl.BlockSpec((D, D), lambda i: (0, 0)),
                  pl.BlockSpec((1, D), lambda i: (0, 0)),
                  pl.BlockSpec((D, D), lambda i: (0, 0)),
                  pl.BlockSpec((1, D), lambda i: (0, 0)),
                  pl.BlockSpec((D, D), lambda i: (0, 0)),
                  pl.BlockSpec((1, D), lambda i: (0, 0))],
        out_specs=pl.BlockSpec((B, D), lambda i: (i, 0)),
        out_shape=jax.ShapeDtypeStruct((AP, D), F32),
    )(atom, w0, b0, w1, b1, wo, bo)


def _sc_mesh():
    return plsc.VectorSubcoreMesh(core_axis_name="c", subcore_axis_name="s",
                                  num_cores=2, num_subcores=16)


def _gather_body(tab_hbm, idx_hbm, out_hbm, idxv, pay, sem):
    c = lax.axis_index("c")
    s = lax.axis_index("s")
    w = s * 2 + c
    base = w * 10240
    pltpu.sync_copy(idx_hbm.at[pl.ds(base, 10240)], idxv)

    def body(j, _):
        pltpu.async_copy(tab_hbm.at[idxv.at[pl.ds(j * 128, 128)]], pay, sem).wait()
        pltpu.sync_copy(pay, out_hbm.at[pl.ds(base + j * 128, 128)])
        return 0

    lax.fori_loop(0, 80, body, 0)


def _gather_call(tab, idx):
    return pl.kernel(
        _gather_body,
        out_type=jax.ShapeDtypeStruct((MP, D), F32),
        mesh=_sc_mesh(),
        scratch_types=[pltpu.VMEM((10240,), I32),
                       pltpu.VMEM((128, D), F32),
                       pltpu.SemaphoreType.DMA],
    )(tab, idx)


def _segsum_msg_body(dst_hbm, msg_hbm, zeros_hbm, out_hbm,
                     dstv, pay, zbig, stg, accum, sem):
    c = lax.axis_index("c")
    s = lax.axis_index("s")
    t0 = s * MSL
    pltpu.sync_copy(dst_hbm.at[pl.ds(t0, MSL)], dstv)
    pltpu.sync_copy(zeros_hbm, zbig)
    zb = s * (ACC // 16)

    for k in range(NCHUNK):
        ck = c * NCHUNK + k
        lo = ck * CHK
        hi = lo + CHK
        pltpu.sync_copy(zbig, accum.at[pl.ds(zb, 128)])
        pltpu.sync_copy(zbig.at[pl.ds(0, 80)], accum.at[pl.ds(zb + 128, 80)])
        plsc.subcore_barrier()

        def pbody(j, _):
            pltpu.async_copy(msg_hbm.at[pl.ds(t0 + j * 128, 128)], pay,
                             sem).wait()
            for q in range(8):
                dv = dstv[pl.ds(j * 128 + q * 16, 16)]
                valid = (dv >= lo) & (dv < hi)
                iv = jnp.where(valid, dv - lo, CHK)
                pltpu.sync_copy(pay.at[pl.ds(q * 16, 16)], accum.at[iv],
                                add=True)
            return 0

        lax.fori_loop(0, MSL // 128, pbody, 0)
        plsc.subcore_barrier()

        wbase = s * (CHK // 16)
        pltpu.sync_copy(accum.at[pl.ds(wbase, 200)], stg)
        pltpu.sync_copy(stg, out_hbm.at[pl.ds(lo + wbase, 200)])
        plsc.subcore_barrier()


def _segsum_msg_call(dst, msg):
    return pl.kernel(
        _segsum_msg_body,
        out_type=jax.ShapeDtypeStruct((E0, D), F32),
        mesh=_sc_mesh(),
        scratch_types=[pltpu.VMEM((MSL,), I32),
                       pltpu.VMEM((128, D), F32),
                       pltpu.VMEM((128, D), F32),
                       pltpu.VMEM((200, D), F32),
                       pltpu.VMEM_SHARED((ACC, D), F32),
                       pltpu.SemaphoreType.DMA],
    )(dst, msg, jnp.zeros((128, D), F32))


def _segsum_atom_body(dst_hbm, a_hbm, zeros_hbm, out_hbm,
                      dstv, pay, zbig, stg, accum):
    c = lax.axis_index("c")
    s = lax.axis_index("s")
    lo = c * ACHK
    t0 = s * ESL
    pltpu.sync_copy(dst_hbm.at[pl.ds(t0, ESL)], dstv)
    pltpu.sync_copy(zeros_hbm, zbig)
    zb = s * (ACA // 16)
    pltpu.sync_copy(zbig, accum.at[pl.ds(zb, 128)])
    pltpu.sync_copy(zbig, accum.at[pl.ds(zb + 128, 128)])
    pltpu.sync_copy(zbig.at[pl.ds(0, 80)], accum.at[pl.ds(zb + 256, 80)])
    plsc.subcore_barrier()

    def pbody(j, _):
        pltpu.sync_copy(a_hbm.at[pl.ds(t0 + j * 128, 128)], pay)
        for q in range(8):
            dv = dstv[pl.ds(j * 128 + q * 16, 16)]
            valid = (dv >= lo) & (dv < lo + ACHK)
            iv = jnp.where(valid, dv - lo, ACHK)
            pltpu.sync_copy(pay.at[pl.ds(q * 16, 16)], accum.at[iv], add=True)
        return 0

    lax.fori_loop(0, ESL // 128, pbody, 0)
    plsc.subcore_barrier()
    wbase = s * (ACHK // 16)
    for p in range(5):
        pltpu.sync_copy(accum.at[pl.ds(wbase + p * 64, 64)], stg)
        pltpu.sync_copy(stg, out_hbm.at[pl.ds(lo + wbase + p * 64, 64)])


def _segsum_atom_call(dst, a):
    return pl.kernel(
        _segsum_atom_body,
        out_type=jax.ShapeDtypeStruct((AP, D), F32),
        mesh=_sc_mesh(),
        scratch_types=[pltpu.VMEM((ESL,), I32),
                       pltpu.VMEM((128, D), F32),
                       pltpu.VMEM((128, D), F32),
                       pltpu.VMEM((64, D), F32),
                       pltpu.VMEM_SHARED((ACA, D), F32)],
    )(dst, a, jnp.zeros((128, D), F32))


def kernel(mji, rbf_ji, sbf_kji, msg_edge_index, edge_index, gate,
           W_src, b_src, W_rbf_mp, W_sbf, W_bil,
           resi_W1, resi_b1, resi_W2, resi_b2, lin_int_W, lin_int_b,
           resm_W1, resm_b1, resm_W2, resm_b2,
           W_rbf_out, dense_W, dense_b, out_W, out_b):
    src_p = jnp.concatenate([msg_edge_index[0],
                             jnp.zeros((MP - M0,), I32)])
    dst_p = jnp.concatenate([msg_edge_index[1],
                             jnp.zeros((MP - M0,), I32)])
    sbf_p = jnp.concatenate([sbf_kji, jnp.zeros((MP - M0, 16), F32)])
    edst_p = jnp.concatenate([edge_index[1], jnp.zeros((EP - E0,), I32)])
    Wb = W_bil.reshape(D, NB * D)

    x = _s1_call(mji, rbf_ji, W_src, b_src.reshape(1, D), W_rbf_mp)
    xk = _gather_call(x, src_p)
    msg = _s3_call(xk, sbf_p, W_sbf, Wb)
    m0 = _segsum_msg_call(dst_p, msg)
    m, a = _s5_call(
        m0, mji, rbf_ji, gate,
        resi_W1, resi_b1.reshape(1, D), resi_W2, resi_b2.reshape(1, D),
        lin_int_W, lin_int_b.reshape(1, D),
        resm_W1[0], resm_b1[0].reshape(1, D), resm_W2[0], resm_b2[0].reshape(1, D),
        resm_W1[1], resm_b1[1].reshape(1, D), resm_W2[1], resm_b2[1].reshape(1, D),
        W_rbf_out)
    a_p = jnp.concatenate([a, jnp.zeros((EP - E0, D), F32)])
    atom_p = _segsum_atom_call(edst_p, a_p)
    out_Wp = jnp.pad(out_W, ((0, D - 1), (0, 0)))
    out_bp = jnp.pad(out_b, (0, D - 1)).reshape(1, D)
    out_p = _s7_call(atom_p, dense_W[0], dense_b[0].reshape(1, D),
                     dense_W[1], dense_b[1].reshape(1, D), out_Wp, out_bp)
    out = out_p[:NA, :1]
    reg = jnp.zeros((), F32)
    return (m, out, reg)


# speedup vs baseline: 2.0247x; 2.0247x over previous
import functools

import jax
import jax.numpy as jnp
from jax import lax
from jax.experimental import pallas as pl
from jax.experimental.pallas import tpu as pltpu
from jax.experimental.pallas import tpu_sc as plsc

F32 = jnp.float32
I32 = jnp.int32

E0 = 160000
M0 = 320000
NA = 10000
D = 128
NB = 8
MP = 327680
EP = 163840
AP = 10240

CKB = 12
CHKP = 1 << CKB
NCKT = 40
ACC2 = 4224
MW = MP // 32
LCAP = 11264
PADV = M0 << CKB

ACHK = 5120
ACA = 5376
ESL = EP // 16


def _act(x):
    return x * (1.0 / (1.0 + jnp.exp(-x)))


def _mm_nt(a, w):
    return lax.dot_general(a, w, (((1,), (1,)), ((), ())),
                           preferred_element_type=F32)


def _s1_body(mji_ref, rbf_ref, ws_ref, bs_ref, wr_ref, x_ref):
    h = _act(_mm_nt(mji_ref[...], ws_ref[...]) + bs_ref[...])
    x_ref[...] = h * _mm_nt(rbf_ref[...], wr_ref[...])


def _s3_body(xk_ref, sbf_ref, wsbf_ref, wb_ref, msg_ref):
    s = _mm_nt(sbf_ref[...], wsbf_ref[...])
    t = jnp.dot(xk_ref[...], wb_ref[...], preferred_element_type=F32)
    acc = t[:, 0:D] * s[:, 0:1]
    for l in range(1, NB):
        acc = acc + t[:, l * D:(l + 1) * D] * s[:, l:l + 1]
    msg_ref[...] = acc


def _s5_body(m0_ref, mji_ref, rbf_ref, gate_ref,
             riW1, rib1, riW2, rib2, liW, lib,
             r1W1, r1b1, r1W2, r1b2, r2W1, r2b1, r2W2, r2b2,
             wro, m_out, a_out):
    def resid(x, W1, b1, W2, b2):
        v = _act(x)
        v = _act(_mm_nt(v, W1[...]) + b1[...])
        v = _mm_nt(v, W2[...]) + b2[...]
        return x + v

    m = m0_ref[...]
    m = resid(m, riW1, rib1, riW2, rib2)
    m = _act(_mm_nt(m, liW[...]) + lib[...]) + gate_ref[...] * mji_ref[...]
    m = resid(m, r1W1, r1b1, r1W2, r1b2)
    m = resid(m, r2W1, r2b1, r2W2, r2b2)
    m_out[...] = m
    a_out[...] = m * _mm_nt(rbf_ref[...], wro[...])


def _s7_body(atom_ref, w0, b0, w1, b1, wo, bo, out_ref):
    h = _act(_mm_nt(atom_ref[...], w0[...]) + b0[...])
    h = _act(_mm_nt(h, w1[...]) + b1[...])
    out_ref[...] = _mm_nt(h, wo[...]) + bo[...]


def _s1_call(mji, rbf, Ws, bs, Wr):
    B = 2000
    n = E0 // B
    return pl.pallas_call(
        _s1_body,
        grid=(n,),
        in_specs=[pl.BlockSpec((B, D), lambda i: (i, 0)),
                  pl.BlockSpec((B, 16), lambda i: (i, 0)),
                  pl.BlockSpec((D, D), lambda i: (0, 0)),
                  pl.BlockSpec((1, D), lambda i: (0, 0)),
                  pl.BlockSpec((D, 16), lambda i: (0, 0))],
        out_specs=pl.BlockSpec((B, D), lambda i: (i, 0)),
        out_shape=jax.ShapeDtypeStruct((E0, D), F32),
    )(mji, rbf, Ws, bs, Wr)


def _s3_call(xk, sbf, Wsbf, Wb):
    B = 512
    n = MP // B
    return pl.pallas_call(
        _s3_body,
        grid=(n,),
        in_specs=[pl.BlockSpec((B, D), lambda i: (i, 0)),
                  pl.BlockSpec((B, 16), lambda i: (i, 0)),
                  pl.BlockSpec((NB, 16), lambda i: (0, 0)),
                  pl.BlockSpec((D, NB * D), lambda i: (0, 0))],
        out_specs=pl.BlockSpec((B, D), lambda i: (i, 0)),
        out_shape=jax.ShapeDtypeStruct((MP, D), F32),
    )(xk, sbf, Wsbf, Wb)


def _s5_call(m0, mji, rbf, gate, riW1, rib1, riW2, rib2, liW, lib,
             r1W1, r1b1, r1W2, r1b2, r2W1, r2b1, r2W2, r2b2, wro):
    B = 2000
    n = E0 // B
    row = lambda: pl.BlockSpec((B, D), lambda i: (i, 0))
    wmat = lambda: pl.BlockSpec((D, D), lambda i: (0, 0))
    wvec = lambda: pl.BlockSpec((1, D), lambda i: (0, 0))
    return pl.pallas_call(
        _s5_body,
        grid=(n,),
        in_specs=[row(), row(), pl.BlockSpec((B, 16), lambda i: (i, 0)), wvec(),
                  wmat(), wvec(), wmat(), wvec(), wmat(), wvec(),
                  wmat(), wvec(), wmat(), wvec(), wmat(), wvec(), wmat(), wvec(),
                  pl.BlockSpec((D, 16), lambda i: (0, 0))],
        out_specs=[row(), row()],
        out_shape=[jax.ShapeDtypeStruct((E0, D), F32),
                   jax.ShapeDtypeStruct((E0, D), F32)],
    )(m0, mji, rbf, gate, riW1, rib1, riW2, rib2, liW, lib,
      r1W1, r1b1, r1W2, r1b2, r2W1, r2b1, r2W2, r2b2, wro)


def _s7_call(atom, w0, b0, w1, b1, wo, bo):
    B = 1280
    n = AP // B
    return pl.pallas_call(
        _s7_body,
        grid=(n,),
        in_specs=[pl.BlockSpec((B, D), lambda i: (i, 0)),
                  pl.BlockSpec((D, D), lambda i: (0, 0)),
                  pl.BlockSpec((1, D), lambda i: (0, 0)),
                  pl.BlockSpec((D, D), lambda i: (0, 0)),
                  pl.BlockSpec((1, D), lambda i: (0, 0)),
                  pl.BlockSpec((D, D), lambda i: (0, 0)),
                  pl.BlockSpec((1, D), lambda i: (0, 0))],
        out_specs=pl.BlockSpec((B, D), lambda i: (i, 0)),
        out_shape=jax.ShapeDtypeStruct((AP, D), F32),
    )(atom, w0, b0, w1, b1, wo, bo)


def _sc_mesh():
    return plsc.VectorSubcoreMesh(core_axis_name="c", subcore_axis_name="s",
                                  num_cores=2, num_subcores=16)


def _gather_body(tab_hbm, idx_hbm, out_hbm, idxv, pay, sem):
    c = lax.axis_index("c")
    s = lax.axis_index("s")
    w = s * 2 + c
    base = w * 10240
    pltpu.sync_copy(idx_hbm.at[pl.ds(base, 10240)], idxv)

    def body(j, _):
        pltpu.async_copy(tab_hbm.at[idxv.at[pl.ds(j * 128, 128)]], pay, sem).wait()
        pltpu.sync_copy(pay, out_hbm.at[pl.ds(base + j * 128, 128)])
        return 0

    lax.fori_loop(0, 80, body, 0)


def _gather_call(tab, idx):
    return pl.kernel(
        _gather_body,
        out_type=jax.ShapeDtypeStruct((MP, D), F32),
        mesh=_sc_mesh(),
        scratch_types=[pltpu.VMEM((10240,), I32),
                       pltpu.VMEM((128, D), F32),
                       pltpu.SemaphoreType.DMA],
    )(tab, idx)


def _bucket_body(dst_hbm, lists_hbm, cnts_hbm, dstv, stage, cnts, smem):
    c = lax.axis_index("c")
    s = lax.axis_index("s")
    w = s * 2 + c
    base = w * MW
    pltpu.sync_copy(dst_hbm.at[pl.ds(base, MW)], dstv)
    lanes = lax.iota(I32, 16)
    for i in range(NCKT):
        smem[i] = jnp.int32(0)

    def mbody(g, _):
        dv = dstv[pl.ds(g * 16, 16)]
        ckv = lax.shift_right_logical(dv, CKB)
        pkv = ((base + g * 16 + lanes) << CKB) | (dv & (CHKP - 1))
        for q in range(16):
            pkq = pkv[q]
            ckq = ckv[q]
            cur = smem[ckq]
            smem[ckq] = cur + 1
            stage[pl.ds(ckq * 640 + (cur & 511), 16)] = jnp.full((16,), pkq, I32)

            @pl.when((cur & 511) == 511)
            def _flush():
                pltpu.sync_copy(
                    stage.at[pl.ds(ckq * 640, 512)],
                    lists_hbm.at[w, ckq, pl.ds((cur >> 9) * 512, 512)])
        return 0

    lax.fori_loop(0, MW // 16, mbody, 0)

    for ck in range(NCKT):
        n = smem[ck]
        npad = (n + 127) & ~127
        padvec = jnp.full((16,), PADV, I32)
        for t in range(8):
            stage[pl.ds(ck * 640 + (n & 511) + t * 16, 16)] = padvec
        pltpu.sync_copy(stage.at[pl.ds(ck * 640, 640)],
                        lists_hbm.at[w, ck, pl.ds((n >> 9) * 512, 640)])
        cnts[pl.ds(ck, 16)] = jnp.full((16,), npad, I32)
    pltpu.sync_copy(cnts.at[pl.ds(0, 64)], cnts_hbm.at[pl.ds(w * 64, 64)])


def _bucket_call(dst):
    return pl.kernel(
        _bucket_body,
        out_type=(jax.ShapeDtypeStruct((32, NCKT, LCAP), I32),
                  jax.ShapeDtypeStruct((2048,), I32)),
        mesh=_sc_mesh(),
        scratch_types=[pltpu.VMEM((MW,), I32),
                       pltpu.VMEM((NCKT * 640,), I32),
                       pltpu.VMEM((64,), I32),
                       pltpu.SMEM((64,), I32)],
    )(dst)


def _accum_body(lists_hbm, cnts_hbm, msg_hbm, zeros_hbm, out_hbm,
                lblk, midb, pay, zbig, stg, cntv, accum, sem):
    c = lax.axis_index("c")
    s = lax.axis_index("s")
    pltpu.sync_copy(cnts_hbm, cntv)
    pltpu.sync_copy(zeros_hbm, zbig)
    zb = s * (ACC2 // 16)

    for k in range(20):
        ck = c * 20 + k
        lo = ck * CHKP
        pltpu.sync_copy(zbig, accum.at[pl.ds(zb, 128)])
        pltpu.sync_copy(zbig, accum.at[pl.ds(zb + 128, 128)])
        pltpu.sync_copy(zbig.at[pl.ds(0, 8)], accum.at[pl.ds(zb + 256, 8)])
        plsc.subcore_barrier()

        for t in range(2):
            w = s * 2 + t
            npad = cntv[pl.ds(w * 64 + ck, 16)][0]
            nblk = lax.shift_right_logical(npad, 7)

            def gbody(j, _):
                pltpu.sync_copy(lists_hbm.at[w, ck, pl.ds(j * 128, 128)], lblk)
                for g in range(8):
                    v = lblk[pl.ds(g * 16, 16)]
                    midb[pl.ds(g * 16, 16)] = lax.shift_right_logical(v, CKB)
                pltpu.async_copy(msg_hbm.at[midb], pay, sem).wait()
                for g in range(8):
                    v = lblk[pl.ds(g * 16, 16)]
                    ldst = v & (CHKP - 1)
                    pltpu.sync_copy(pay.at[pl.ds(g * 16, 16)],
                                    accum.at[ldst], add=True)
                return 0

            lax.fori_loop(0, nblk, gbody, 0)
        plsc.subcore_barrier()

        wbase = s * (CHKP // 16)

        @pl.when(lo + wbase < E0)
        def _writeout():
            for p in range(2):
                pltpu.sync_copy(accum.at[pl.ds(wbase + p * 128, 128)], stg)
                pltpu.sync_copy(stg,
                                out_hbm.at[pl.ds(lo + wbase + p * 128, 128)])
        plsc.subcore_barrier()


def _segsum_msg_call(dst, msg):
    lists, cnts = _bucket_call(dst)
    return pl.kernel(
        _accum_body,
        out_type=jax.ShapeDtypeStruct((E0, D), F32),
        mesh=_sc_mesh(),
        scratch_types=[pltpu.VMEM((128,), I32),
                       pltpu.VMEM((128,), I32),
                       pltpu.VMEM((128, D), F32),
                       pltpu.VMEM((128, D), F32),
                       pltpu.VMEM((128, D), F32),
                       pltpu.VMEM((2048,), I32),
                       pltpu.VMEM_SHARED((ACC2, D), F32),
                       pltpu.SemaphoreType.DMA],
    )(lists, cnts, msg, jnp.zeros((128, D), F32))


def _segsum_atom_body(dst_hbm, a_hbm, zeros_hbm, out_hbm,
                      dstv, pay, zbig, stg, accum):
    c = lax.axis_index("c")
    s = lax.axis_index("s")
    lo = c * ACHK
    t0 = s * ESL
    pltpu.sync_copy(dst_hbm.at[pl.ds(t0, ESL)], dstv)
    pltpu.sync_copy(zeros_hbm, zbig)
    zb = s * (ACA // 16)
    pltpu.sync_copy(zbig, accum.at[pl.ds(zb, 128)])
    pltpu.sync_copy(zbig, accum.at[pl.ds(zb + 128, 128)])
    pltpu.sync_copy(zbig.at[pl.ds(0, 80)], accum.at[pl.ds(zb + 256, 80)])
    plsc.subcore_barrier()

    def pbody(j, _):
        pltpu.sync_copy(a_hbm.at[pl.ds(t0 + j * 128, 128)], pay)
        for q in range(8):
            dv = dstv[pl.ds(j * 128 + q * 16, 16)]
            valid = (dv >= lo) & (dv < lo + ACHK)
            iv = jnp.where(valid, dv - lo, ACHK)
            pltpu.sync_copy(pay.at[pl.ds(q * 16, 16)], accum.at[iv], add=True)
        return 0

    lax.fori_loop(0, ESL // 128, pbody, 0)
    plsc.subcore_barrier()
    wbase = s * (ACHK // 16)
    for p in range(5):
        pltpu.sync_copy(accum.at[pl.ds(wbase + p * 64, 64)], stg)
        pltpu.sync_copy(stg, out_hbm.at[pl.ds(lo + wbase + p * 64, 64)])


def _segsum_atom_call(dst, a):
    return pl.kernel(
        _segsum_atom_body,
        out_type=jax.ShapeDtypeStruct((AP, D), F32),
        mesh=_sc_mesh(),
        scratch_types=[pltpu.VMEM((ESL,), I32),
                       pltpu.VMEM((128, D), F32),
                       pltpu.VMEM((128, D), F32),
                       pltpu.VMEM((64, D), F32),
                       pltpu.VMEM_SHARED((ACA, D), F32)],
    )(dst, a, jnp.zeros((128, D), F32))


def kernel(mji, rbf_ji, sbf_kji, msg_edge_index, edge_index, gate,
           W_src, b_src, W_rbf_mp, W_sbf, W_bil,
           resi_W1, resi_b1, resi_W2, resi_b2, lin_int_W, lin_int_b,
           resm_W1, resm_b1, resm_W2, resm_b2,
           W_rbf_out, dense_W, dense_b, out_W, out_b):
    src_p = jnp.concatenate([msg_edge_index[0],
                             jnp.zeros((MP - M0,), I32)])
    dst_p = jnp.concatenate([msg_edge_index[1],
                             jnp.zeros((MP - M0,), I32)])
    sbf_p = jnp.concatenate([sbf_kji, jnp.zeros((MP - M0, 16), F32)])
    edst_p = jnp.concatenate([edge_index[1], jnp.zeros((EP - E0,), I32)])
    Wb = W_bil.reshape(D, NB * D)

    x = _s1_call(mji, rbf_ji, W_src, b_src.reshape(1, D), W_rbf_mp)
    xk = _gather_call(x, src_p)
    msg = _s3_call(xk, sbf_p, W_sbf, Wb)
    m0 = _segsum_msg_call(dst_p, msg)
    m, a = _s5_call(
        m0, mji, rbf_ji, gate,
        resi_W1, resi_b1.reshape(1, D), resi_W2, resi_b2.reshape(1, D),
        lin_int_W, lin_int_b.reshape(1, D),
        resm_W1[0], resm_b1[0].reshape(1, D), resm_W2[0], resm_b2[0].reshape(1, D),
        resm_W1[1], resm_b1[1].reshape(1, D), resm_W2[1], resm_b2[1].reshape(1, D),
        W_rbf_out)
    a_p = jnp.concatenate([a, jnp.zeros((EP - E0, D), F32)])
    atom_p = _segsum_atom_call(edst_p, a_p)
    out_Wp = jnp.pad(out_W, ((0, D - 1), (0, 0)))
    out_bp = jnp.pad(out_b, (0, D - 1)).reshape(1, D)
    out_p = _s7_call(atom_p, dense_W[0], dense_b[0].reshape(1, D),
                     dense_W[1], dense_b[1].reshape(1, D), out_Wp, out_bp)
    out = out_p[:NA, :1]
    reg = jnp.zeros((), F32)
    return (m, out, reg)


# speedup vs baseline: 2.0661x; 1.0204x over previous
import functools

import jax
import jax.numpy as jnp
from jax import lax
from jax.experimental import pallas as pl
from jax.experimental.pallas import tpu as pltpu
from jax.experimental.pallas import tpu_sc as plsc

F32 = jnp.float32
I32 = jnp.int32

E0 = 160000
M0 = 320000
NA = 10000
D = 128
NB = 8
MP = 327680
EP = 163840
AP = 10240

CKB = 12
CHKP = 1 << CKB
NCKT = 40
ACC2 = 4224
MW = MP // 32
LCAP = 11264
PADV = M0 << CKB

ACHK = 5120
ACA = 5376
ESL = EP // 16


def _act(x):
    return x * (1.0 / (1.0 + jnp.exp(-x)))


def _mm_nt(a, w):
    return lax.dot_general(a, w, (((1,), (1,)), ((), ())),
                           preferred_element_type=F32)


def _s1_body(mji_ref, rbf_ref, ws_ref, bs_ref, wr_ref, x_ref):
    h = _act(_mm_nt(mji_ref[...], ws_ref[...]) + bs_ref[...])
    x_ref[...] = h * _mm_nt(rbf_ref[...], wr_ref[...])


def _s3_body(xk_ref, sbf_ref, wsbf_ref, wb_ref, msg_ref):
    s = _mm_nt(sbf_ref[...], wsbf_ref[...])
    t = jnp.dot(xk_ref[...], wb_ref[...], preferred_element_type=F32)
    acc = t[:, 0:D] * s[:, 0:1]
    for l in range(1, NB):
        acc = acc + t[:, l * D:(l + 1) * D] * s[:, l:l + 1]
    msg_ref[...] = acc


def _s5_body(m0_ref, mji_ref, rbf_ref, gate_ref,
             riW1, rib1, riW2, rib2, liW, lib,
             r1W1, r1b1, r1W2, r1b2, r2W1, r2b1, r2W2, r2b2,
             wro, m_out, a_out):
    def resid(x, W1, b1, W2, b2):
        v = _act(x)
        v = _act(_mm_nt(v, W1[...]) + b1[...])
        v = _mm_nt(v, W2[...]) + b2[...]
        return x + v

    m = m0_ref[...]
    m = resid(m, riW1, rib1, riW2, rib2)
    m = _act(_mm_nt(m, liW[...]) + lib[...]) + gate_ref[...] * mji_ref[...]
    m = resid(m, r1W1, r1b1, r1W2, r1b2)
    m = resid(m, r2W1, r2b1, r2W2, r2b2)
    m_out[...] = m
    a_out[...] = m * _mm_nt(rbf_ref[...], wro[...])


def _s7_body(atom_ref, w0, b0, w1, b1, wo, bo, out_ref):
    h = _act(_mm_nt(atom_ref[...], w0[...]) + b0[...])
    h = _act(_mm_nt(h, w1[...]) + b1[...])
    out_ref[...] = _mm_nt(h, wo[...]) + bo[...]


def _s1_call(mji, rbf, Ws, bs, Wr):
    B = 2000
    n = E0 // B
    return pl.pallas_call(
        _s1_body,
        grid=(n,),
        in_specs=[pl.BlockSpec((B, D), lambda i: (i, 0)),
                  pl.BlockSpec((B, 16), lambda i: (i, 0)),
                  pl.BlockSpec((D, D), lambda i: (0, 0)),
                  pl.BlockSpec((1, D), lambda i: (0, 0)),
                  pl.BlockSpec((D, 16), lambda i: (0, 0))],
        out_specs=pl.BlockSpec((B, D), lambda i: (i, 0)),
        out_shape=jax.ShapeDtypeStruct((E0, D), F32),
    )(mji, rbf, Ws, bs, Wr)


def _s3_call(xk, sbf, Wsbf, Wb):
    B = 512
    n = MP // B
    return pl.pallas_call(
        _s3_body,
        grid=(n,),
        in_specs=[pl.BlockSpec((B, D), lambda i: (i, 0)),
                  pl.BlockSpec((B, 16), lambda i: (i, 0)),
                  pl.BlockSpec((NB, 16), lambda i: (0, 0)),
                  pl.BlockSpec((D, NB * D), lambda i: (0, 0))],
        out_specs=pl.BlockSpec((B, D), lambda i: (i, 0)),
        out_shape=jax.ShapeDtypeStruct((MP, D), F32),
    )(xk, sbf, Wsbf, Wb)


def _s5_call(m0, mji, rbf, gate, riW1, rib1, riW2, rib2, liW, lib,
             r1W1, r1b1, r1W2, r1b2, r2W1, r2b1, r2W2, r2b2, wro):
    B = 2000
    n = E0 // B
    row = lambda: pl.BlockSpec((B, D), lambda i: (i, 0))
    wmat = lambda: pl.BlockSpec((D, D), lambda i: (0, 0))
    wvec = lambda: pl.BlockSpec((1, D), lambda i: (0, 0))
    return pl.pallas_call(
        _s5_body,
        grid=(n,),
        in_specs=[row(), row(), pl.BlockSpec((B, 16), lambda i: (i, 0)), wvec(),
                  wmat(), wvec(), wmat(), wvec(), wmat(), wvec(),
                  wmat(), wvec(), wmat(), wvec(), wmat(), wvec(), wmat(), wvec(),
                  pl.BlockSpec((D, 16), lambda i: (0, 0))],
        out_specs=[row(), row()],
        out_shape=[jax.ShapeDtypeStruct((E0, D), F32),
                   jax.ShapeDtypeStruct((E0, D), F32)],
    )(m0, mji, rbf, gate, riW1, rib1, riW2, rib2, liW, lib,
      r1W1, r1b1, r1W2, r1b2, r2W1, r2b1, r2W2, r2b2, wro)


def _s7_call(atom, w0, b0, w1, b1, wo, bo):
    B = 1280
    n = AP // B
    return pl.pallas_call(
        _s7_body,
        grid=(n,),
        in_specs=[pl.BlockSpec((B, D), lambda i: (i, 0)),
                  pl.BlockSpec((D, D), lambda i: (0, 0)),
                  pl.BlockSpec((1, D), lambda i: (0, 0)),
                  pl.BlockSpec((D, D), lambda i: (0, 0)),
                  pl.BlockSpec((1, D), lambda i: (0, 0)),
                  pl.BlockSpec((D, D), lambda i: (0, 0)),
                  pl.BlockSpec((1, D), lambda i: (0, 0))],
        out_specs=pl.BlockSpec((B, D), lambda i: (i, 0)),
        out_shape=jax.ShapeDtypeStruct((AP, D), F32),
    )(atom, w0, b0, w1, b1, wo, bo)


def _sc_mesh():
    return plsc.VectorSubcoreMesh(core_axis_name="c", subcore_axis_name="s",
                                  num_cores=2, num_subcores=16)


def _gather_body(tab_hbm, idx_hbm, out_hbm, idxv, pay, sem):
    c = lax.axis_index("c")
    s = lax.axis_index("s")
    w = s * 2 + c
    base = w * 10240
    pltpu.sync_copy(idx_hbm.at[w], idxv)

    def body(j, _):
        cps = [pltpu.async_copy(tab_hbm.at[idxv.at[j, r]],
                                pay.at[pl.ds(r * 128, 128)], sem)
               for r in range(4)]
        for cp in cps:
            cp.wait()
        pltpu.sync_copy(pay, out_hbm.at[pl.ds(base + j * 512, 512)])
        return 0

    lax.fori_loop(0, 20, body, 0)


def _gather_call(tab, idx):
    return pl.kernel(
        _gather_body,
        out_type=jax.ShapeDtypeStruct((MP, D), F32),
        mesh=_sc_mesh(),
        scratch_types=[pltpu.VMEM((20, 4, 128), I32),
                       pltpu.VMEM((512, D), F32),
                       pltpu.SemaphoreType.DMA],
    )(tab, idx)


def _bucket_body(dst_hbm, lists_hbm, cnts_hbm, dstv, stage, cnts, smem):
    c = lax.axis_index("c")
    s = lax.axis_index("s")
    w = s * 2 + c
    base = w * MW
    pltpu.sync_copy(dst_hbm.at[pl.ds(base, MW)], dstv)
    lanes = lax.iota(I32, 16)
    for i in range(NCKT):
        smem[i] = jnp.int32(0)

    def mbody(g, _):
        dv = dstv[pl.ds(g * 16, 16)]
        ckv = lax.shift_right_logical(dv, CKB)
        pkv = ((base + g * 16 + lanes) << CKB) | (dv & (CHKP - 1))
        for q in range(16):
            pkq = pkv[q]
            ckq = ckv[q]
            cur = smem[ckq]
            smem[ckq] = cur + 1
            stage[pl.ds(ckq * 640 + (cur & 511), 16)] = jnp.full((16,), pkq, I32)

            @pl.when((cur & 511) == 511)
            def _flush():
                pltpu.sync_copy(
                    stage.at[pl.ds(ckq * 640, 512)],
                    lists_hbm.at[w, ckq, pl.ds((cur >> 9) * 512, 512)])
        return 0

    lax.fori_loop(0, MW // 16, mbody, 0)

    for ck in range(NCKT):
        n = smem[ck]
        npad = (n + 127) & ~127
        padvec = jnp.full((16,), PADV, I32)
        for t in range(8):
            stage[pl.ds(ck * 640 + (n & 511) + t * 16, 16)] = padvec
        pltpu.sync_copy(stage.at[pl.ds(ck * 640, 640)],
                        lists_hbm.at[w, ck, pl.ds((n >> 9) * 512, 640)])
        cnts[pl.ds(ck, 16)] = jnp.full((16,), npad, I32)
    pltpu.sync_copy(cnts.at[pl.ds(0, 64)], cnts_hbm.at[pl.ds(w * 64, 64)])


def _bucket_call(dst):
    return pl.kernel(
        _bucket_body,
        out_type=(jax.ShapeDtypeStruct((32, NCKT, LCAP), I32),
                  jax.ShapeDtypeStruct((2048,), I32)),
        mesh=_sc_mesh(),
        scratch_types=[pltpu.VMEM((MW,), I32),
                       pltpu.VMEM((NCKT * 640,), I32),
                       pltpu.VMEM((64,), I32),
                       pltpu.SMEM((64,), I32)],
    )(dst)


def _accum_body(lists_hbm, cnts_hbm, msg_hbm, zeros_hbm, out_hbm,
                lblk, midb, ldstb, pay, zbig, cntv, accum, sem, sem2):
    c = lax.axis_index("c")
    s = lax.axis_index("s")
    pltpu.sync_copy(cnts_hbm, cntv)
    pltpu.sync_copy(zeros_hbm, zbig)
    zb = s * (ACC2 // 16)

    def chunk_body(k, _carry):
        ck = c * 20 + k
        lo = ck * CHKP
        pltpu.sync_copy(zbig, accum.at[pl.ds(zb, 128)])
        pltpu.sync_copy(zbig, accum.at[pl.ds(zb + 128, 128)])
        pltpu.sync_copy(zbig.at[pl.ds(0, 8)], accum.at[pl.ds(zb + 256, 8)])
        plsc.subcore_barrier()

        for t in range(2):
            w = s * 2 + t
            npad = cntv[pl.ds(w * 64 + ck, 16)][0]
            nsb = lax.shift_right_logical(npad, 9)
            ntail = lax.shift_right_logical(npad, 7) & 3

            def unpack512():
                for g in range(32):
                    v = lblk[pl.ds(g * 16, 16)]
                    midb[g // 8, pl.ds((g % 8) * 16, 16)] = (
                        lax.shift_right_logical(v, CKB))
                    ldstb[g // 8, pl.ds((g % 8) * 16, 16)] = v & (CHKP - 1)

            def sbody(jj, _):
                pltpu.sync_copy(lists_hbm.at[w, ck, pl.ds(jj * 512, 512)], lblk)
                unpack512()
                cps = [pltpu.async_copy(msg_hbm.at[midb.at[r]],
                                        pay.at[pl.ds(r * 128, 128)], sem)
                       for r in range(4)]
                for cp in cps:
                    cp.wait()
                cps = [pltpu.async_copy(pay.at[pl.ds(r * 128, 128)],
                                        accum.at[ldstb.at[r]], sem2, add=True)
                       for r in range(4)]
                for cp in cps:
                    cp.wait()
                return 0

            lax.fori_loop(0, nsb, sbody, 0)

            @pl.when(ntail > 0)
            def _tail():
                pltpu.sync_copy(lists_hbm.at[w, ck, pl.ds(nsb * 512, 512)],
                                lblk)
                unpack512()
                for r in range(3):
                    @pl.when(r < ntail)
                    def _one():
                        pltpu.async_copy(msg_hbm.at[midb.at[r]],
                                         pay.at[pl.ds(r * 128, 128)],
                                         sem).wait()
                        pltpu.async_copy(pay.at[pl.ds(r * 128, 128)],
                                         accum.at[ldstb.at[r]], sem2,
                                         add=True).wait()
        plsc.subcore_barrier()

        wbase = s * (CHKP // 16)

        @pl.when(lo + wbase < E0)
        def _writeout():
            for p in range(2):
                st = pay.at[pl.ds(p * 128, 128)]
                pltpu.sync_copy(accum.at[pl.ds(wbase + p * 128, 128)], st)
                pltpu.sync_copy(st,
                                out_hbm.at[pl.ds(lo + wbase + p * 128, 128)])
        plsc.subcore_barrier()
        return 0

    lax.fori_loop(0, 20, chunk_body, 0)


def _segsum_msg_call(dst, msg):
    lists, cnts = _bucket_call(dst)
    return pl.kernel(
        _accum_body,
        out_type=jax.ShapeDtypeStruct((E0, D), F32),
        mesh=_sc_mesh(),
        scratch_types=[pltpu.VMEM((512,), I32),
                       pltpu.VMEM((4, 128), I32),
                       pltpu.VMEM((4, 128), I32),
                       pltpu.VMEM((512, D), F32),
                       pltpu.VMEM((128, D), F32),
                       pltpu.VMEM((2048,), I32),
                       pltpu.VMEM_SHARED((ACC2, D), F32),
                       pltpu.SemaphoreType.DMA,
                       pltpu.SemaphoreType.DMA],
    )(lists, cnts, msg, jnp.zeros((128, D), F32))


def _segsum_atom_body(dst_hbm, a_hbm, zeros_hbm, out_hbm,
                      dstv, ldstb, pay, zbig, stg, accum):
    c = lax.axis_index("c")
    s = lax.axis_index("s")
    lo = c * ACHK
    t0 = s * ESL
    pltpu.sync_copy(dst_hbm.at[pl.ds(t0, ESL)], dstv)
    pltpu.sync_copy(zeros_hbm, zbig)
    zb = s * (ACA // 16)
    pltpu.sync_copy(zbig, accum.at[pl.ds(zb, 128)])
    pltpu.sync_copy(zbig, accum.at[pl.ds(zb + 128, 128)])
    pltpu.sync_copy(zbig.at[pl.ds(0, 80)], accum.at[pl.ds(zb + 256, 80)])
    plsc.subcore_barrier()

    def pbody(j, _):
        pltpu.sync_copy(a_hbm.at[pl.ds(t0 + j * 128, 128)], pay)
        for q in range(8):
            dv = dstv[pl.ds(j * 128 + q * 16, 16)]
            valid = (dv >= lo) & (dv < lo + ACHK)
            ldstb[pl.ds(q * 16, 16)] = jnp.where(valid, dv - lo, ACHK)
        pltpu.sync_copy(pay, accum.at[ldstb], add=True)
        return 0

    lax.fori_loop(0, ESL // 128, pbody, 0)
    plsc.subcore_barrier()
    wbase = s * (ACHK // 16)
    for p in range(5):
        pltpu.sync_copy(accum.at[pl.ds(wbase + p * 64, 64)], stg)
        pltpu.sync_copy(stg, out_hbm.at[pl.ds(lo + wbase + p * 64, 64)])


def _segsum_atom_call(dst, a):
    return pl.kernel(
        _segsum_atom_body,
        out_type=jax.ShapeDtypeStruct((AP, D), F32),
        mesh=_sc_mesh(),
        scratch_types=[pltpu.VMEM((ESL,), I32),
                       pltpu.VMEM((128,), I32),
                       pltpu.VMEM((128, D), F32),
                       pltpu.VMEM((128, D), F32),
                       pltpu.VMEM((64, D), F32),
                       pltpu.VMEM_SHARED((ACA, D), F32)],
    )(dst, a, jnp.zeros((128, D), F32))


def kernel(mji, rbf_ji, sbf_kji, msg_edge_index, edge_index, gate,
           W_src, b_src, W_rbf_mp, W_sbf, W_bil,
           resi_W1, resi_b1, resi_W2, resi_b2, lin_int_W, lin_int_b,
           resm_W1, resm_b1, resm_W2, resm_b2,
           W_rbf_out, dense_W, dense_b, out_W, out_b):
    src_p = jnp.concatenate([msg_edge_index[0],
                             jnp.zeros((MP - M0,), I32)])
    dst_p = jnp.concatenate([msg_edge_index[1],
                             jnp.zeros((MP - M0,), I32)])
    sbf_p = jnp.concatenate([sbf_kji, jnp.zeros((MP - M0, 16), F32)])
    edst_p = jnp.concatenate([edge_index[1], jnp.zeros((EP - E0,), I32)])
    Wb = W_bil.reshape(D, NB * D)

    x = _s1_call(mji, rbf_ji, W_src, b_src.reshape(1, D), W_rbf_mp)
    xk = _gather_call(x, src_p.reshape(32, 20, 4, 128))
    msg = _s3_call(xk, sbf_p, W_sbf, Wb)
    m0 = _segsum_msg_call(dst_p, msg)
    m, a = _s5_call(
        m0, mji, rbf_ji, gate,
        resi_W1, resi_b1.reshape(1, D), resi_W2, resi_b2.reshape(1, D),
        lin_int_W, lin_int_b.reshape(1, D),
        resm_W1[0], resm_b1[0].reshape(1, D), resm_W2[0], resm_b2[0].reshape(1, D),
        resm_W1[1], resm_b1[1].reshape(1, D), resm_W2[1], resm_b2[1].reshape(1, D),
        W_rbf_out)
    a_p = jnp.concatenate([a, jnp.zeros((EP - E0, D), F32)])
    atom_p = _segsum_atom_call(edst_p, a_p)
    out_Wp = jnp.pad(out_W, ((0, D - 1), (0, 0)))
    out_bp = jnp.pad(out_b, (0, D - 1)).reshape(1, D)
    out_p = _s7_call(atom_p, dense_W[0], dense_b[0].reshape(1, D),
                     dense_W[1], dense_b[1].reshape(1, D), out_Wp, out_bp)
    out = out_p[:NA, :1]
    reg = jnp.zeros((), F32)
    return (m, out, reg)
